# baseline (speedup 1.0000x reference)
import functools

import jax
import jax.numpy as jnp
from jax import lax
from jax.experimental import pallas as pl
from jax.experimental.pallas import tpu as pltpu
from jax.experimental.pallas import tpu_sc as plsc

_L = 16


def kernel(sent_group_scores, sel_sent_emb, sel_sent_masks, group_embs, candi_sent_masks, bias):
    B, S = sent_group_scores.shape
    n = B * S
    info = plsc.get_sparse_core_info()
    nc, ns = info.num_cores, info.num_subcores
    nw = nc * ns
    per_w = n // nw

    scores_flat = jnp.reshape(sent_group_scores, (n,))
    mask_flat = jnp.reshape(candi_sent_masks, (n,))
    bias_vec = jnp.broadcast_to(jnp.asarray(bias, jnp.float32), (_L,))

    mesh = plsc.VectorSubcoreMesh(core_axis_name="c", subcore_axis_name="s")

    @functools.partial(
        pl.kernel,
        mesh=mesh,
        out_type=jax.ShapeDtypeStruct((n,), jnp.float32),
        scratch_types=[
            pltpu.VMEM((per_w,), jnp.float32),
            pltpu.VMEM((per_w,), jnp.int32),
            pltpu.VMEM((_L,), jnp.float32),
        ],
    )
    def sc_fn(scores_hbm, mask_hbm, bias_hbm, out_hbm, scores_v, mask_v, bias_v):
        wid = lax.axis_index("s") * nc + lax.axis_index("c")
        base = wid * per_w
        pltpu.sync_copy(scores_hbm.at[pl.ds(base, per_w)], scores_v)
        pltpu.sync_copy(mask_hbm.at[pl.ds(base, per_w)], mask_v)
        pltpu.sync_copy(bias_hbm, bias_v)
        b = bias_v[...]
        for i in range(per_w // _L):
            sl = pl.ds(i * _L, _L)
            scores_v[sl] = (scores_v[sl] + b) * mask_v[sl].astype(jnp.float32)
        pltpu.sync_copy(scores_v, out_hbm.at[pl.ds(base, per_w)])

    out_flat = sc_fn(scores_flat, mask_flat, bias_vec)
    return jnp.reshape(out_flat, (B, S))


# speedup vs baseline: 1.0181x; 1.0181x over previous
import functools

import jax
import jax.numpy as jnp
from jax import lax
from jax.experimental import pallas as pl
from jax.experimental.pallas import tpu as pltpu
from jax.experimental.pallas import tpu_sc as plsc

_L = 16


def kernel(sent_group_scores, sel_sent_emb, sel_sent_masks, group_embs, candi_sent_masks, bias):
    B, S = sent_group_scores.shape
    n = B * S
    info = plsc.get_sparse_core_info()
    nc, ns = info.num_cores, info.num_subcores
    nw = nc * ns
    per_w = n // nw

    scores_flat = jnp.reshape(sent_group_scores, (n,))
    mask_flat = jnp.reshape(candi_sent_masks, (n,))
    bias_vec = jnp.broadcast_to(jnp.asarray(bias, jnp.float32), (_L,))

    mesh = plsc.VectorSubcoreMesh(core_axis_name="c", subcore_axis_name="s")

    @functools.partial(
        pl.kernel,
        mesh=mesh,
        out_type=jax.ShapeDtypeStruct((n,), jnp.float32),
        scratch_types=[
            pltpu.VMEM((per_w,), jnp.float32),
            pltpu.VMEM((per_w,), jnp.int32),
            pltpu.VMEM((_L,), jnp.float32),
            pltpu.SemaphoreType.DMA,
        ],
    )
    def sc_fn(scores_hbm, mask_hbm, bias_hbm, out_hbm, scores_v, mask_v, bias_v, sem):
        wid = lax.axis_index("s") * nc + lax.axis_index("c")
        base = wid * per_w
        h1 = pltpu.async_copy(scores_hbm.at[pl.ds(base, per_w)], scores_v, sem)
        h2 = pltpu.async_copy(mask_hbm.at[pl.ds(base, per_w)], mask_v, sem)
        h3 = pltpu.async_copy(bias_hbm, bias_v, sem)
        h1.wait()
        h2.wait()
        h3.wait()
        b = bias_v[...]

        def body(i, carry):
            sl = pl.ds(i * _L, _L)
            scores_v[sl] = (scores_v[sl] + b) * mask_v[sl].astype(jnp.float32)
            return carry

        lax.fori_loop(0, per_w // _L, body, 0)
        pltpu.sync_copy(scores_v, out_hbm.at[pl.ds(base, per_w)])

    out_flat = sc_fn(scores_flat, mask_flat, bias_vec)
    return jnp.reshape(out_flat, (B, S))


# speedup vs baseline: 1.1138x; 1.0940x over previous
import functools

import jax
import jax.numpy as jnp
from jax import lax
from jax.experimental import pallas as pl
from jax.experimental.pallas import tpu as pltpu
from jax.experimental.pallas import tpu_sc as plsc

_L = 16


def kernel(sent_group_scores, sel_sent_emb, sel_sent_masks, group_embs, candi_sent_masks, bias):
    B, S = sent_group_scores.shape
    n = B * S
    info = plsc.get_sparse_core_info()
    nc, ns = 1, info.num_subcores
    nw = nc * ns
    per_w = n // nw

    scores_flat = jnp.reshape(sent_group_scores, (n,))
    mask_flat = jnp.reshape(candi_sent_masks, (n,))
    bias_vec = jnp.broadcast_to(jnp.asarray(bias, jnp.float32), (_L,))

    mesh = plsc.VectorSubcoreMesh(core_axis_name="c", subcore_axis_name="s", num_cores=nc)

    @functools.partial(
        pl.kernel,
        mesh=mesh,
        out_type=jax.ShapeDtypeStruct((n,), jnp.float32),
        scratch_types=[
            pltpu.VMEM((per_w,), jnp.float32),
            pltpu.VMEM((per_w,), jnp.int32),
            pltpu.VMEM((_L,), jnp.float32),
            pltpu.SemaphoreType.DMA,
        ],
    )
    def sc_fn(scores_hbm, mask_hbm, bias_hbm, out_hbm, scores_v, mask_v, bias_v, sem):
        wid = lax.axis_index("s") * nc + lax.axis_index("c")
        base = wid * per_w
        h1 = pltpu.async_copy(scores_hbm.at[pl.ds(base, per_w)], scores_v, sem)
        h2 = pltpu.async_copy(mask_hbm.at[pl.ds(base, per_w)], mask_v, sem)
        h3 = pltpu.async_copy(bias_hbm, bias_v, sem)
        h1.wait()
        h2.wait()
        h3.wait()
        b = bias_v[...]

        def body(i, carry):
            sl = pl.ds(i * _L, _L)
            scores_v[sl] = (scores_v[sl] + b) * mask_v[sl].astype(jnp.float32)
            return carry

        lax.fori_loop(0, per_w // _L, body, 0)
        pltpu.sync_copy(scores_v, out_hbm.at[pl.ds(base, per_w)])

    out_flat = sc_fn(scores_flat, mask_flat, bias_vec)
    return jnp.reshape(out_flat, (B, S))


# speedup vs baseline: 1.1381x; 1.0218x over previous
import functools

import jax
import jax.numpy as jnp
from jax import lax
from jax.experimental import pallas as pl
from jax.experimental.pallas import tpu as pltpu
from jax.experimental.pallas import tpu_sc as plsc

_L = 16


def kernel(sent_group_scores, sel_sent_emb, sel_sent_masks, group_embs, candi_sent_masks, bias):
    B, S = sent_group_scores.shape
    n = B * S
    info = plsc.get_sparse_core_info()
    nc, ns = 1, info.num_subcores
    nw = nc * ns
    per_w = n // nw

    scores_flat = jnp.reshape(sent_group_scores, (n,))
    mask_flat = jnp.reshape(candi_sent_masks, (n,))
    bias_vec = jnp.broadcast_to(jnp.asarray(bias, jnp.float32), (_L,))

    mesh = plsc.VectorSubcoreMesh(core_axis_name="c", subcore_axis_name="s", num_cores=nc)

    @functools.partial(
        pl.kernel,
        mesh=mesh,
        out_type=jax.ShapeDtypeStruct((n,), jnp.float32),
        scratch_types=[
            pltpu.VMEM((per_w,), jnp.float32),
            pltpu.VMEM((per_w,), jnp.int32),
            pltpu.VMEM((_L,), jnp.float32),
            pltpu.SemaphoreType.DMA,
        ],
    )
    def sc_fn(scores_hbm, mask_hbm, bias_hbm, out_hbm, scores_v, mask_v, bias_v, sem):
        wid = lax.axis_index("s") * nc + lax.axis_index("c")
        base = wid * per_w
        h1 = pltpu.async_copy(scores_hbm.at[pl.ds(base, per_w)], scores_v, sem)
        h2 = pltpu.async_copy(mask_hbm.at[pl.ds(base, per_w)], mask_v, sem)
        h3 = pltpu.async_copy(bias_hbm, bias_v, sem)
        h1.wait()
        h2.wait()
        h3.wait()
        b = bias_v[...]

        unroll = 4

        def body(i, carry):
            base_i = i * (_L * unroll)
            for j in range(unroll):
                sl = pl.ds(base_i + j * _L, _L)
                scores_v[sl] = (scores_v[sl] + b) * mask_v[sl].astype(jnp.float32)
            return carry

        lax.fori_loop(0, per_w // (_L * unroll), body, 0)
        pltpu.sync_copy(scores_v, out_hbm.at[pl.ds(base, per_w)])

    out_flat = sc_fn(scores_flat, mask_flat, bias_vec)
    return jnp.reshape(out_flat, (B, S))
